# baseline (speedup 1.0000x reference)
import functools

import jax
import jax.numpy as jnp
from jax import lax
from jax.experimental import pallas as pl
from jax.experimental.pallas import tpu as pltpu
from jax.experimental.pallas import tpu_sc as plsc

_AXES_LENS = (1536, 512, 512)
_CW = (32, 48, 48)
_COFF = (0, 32, 80)
_OUT_W = 128

_NC = 2
_NS = 16
_NW = _NC * _NS
_CHUNK = 128
_NBUF = 2


def _tables_body(a0, a1, a2, o0, o1, o2):
    for a, o, off, w in zip((a0, a1, a2), (o0, o1, o2), _COFF, _CW):
        x = a[...]
        col = lax.broadcasted_iota(jnp.int32, x.shape, 1)
        val = jnp.where((col & 1) == 0, jnp.cos(x), jnp.sin(x))
        in_range = (col >= off) & (col < off + w)
        o[...] = jnp.where(in_range, val, 0.0)


def _make_ptabs(freqs0, freqs1, freqs2):
    reps = []
    for f, off, w in zip((freqs0, freqs1, freqs2), _COFF, _CW):
        r = jnp.repeat(f, 2, axis=1)
        reps.append(jnp.pad(r, ((0, 0), (off, _OUT_W - off - w))))
    out_shape = [
        jax.ShapeDtypeStruct((_AXES_LENS[i], _OUT_W), jnp.float32)
        for i in range(3)
    ]
    return pl.pallas_call(_tables_body, out_shape=out_shape)(*reps)


def _gather_body(ptab0, ptab1, ptab2, idx0, idx1, idx2, out,
                 iv0, iv1, iv2, bigs, gsem, ssem):
    wid = lax.axis_index("s") * _NC + lax.axis_index("c")
    per_w = iv0.shape[0]
    n_chunks = per_w // _CHUNK
    base = wid * per_w
    pltpu.sync_copy(idx0.at[pl.ds(base, per_w)], iv0)
    pltpu.sync_copy(idx1.at[pl.ds(base, per_w)], iv1)
    pltpu.sync_copy(idx2.at[pl.ds(base, per_w)], iv2)
    stores = [None] * _NBUF
    for j in range(n_chunks):
        b = j % _NBUF
        if stores[b] is not None:
            stores[b].wait()
        sl = pl.ds(j * _CHUNK, _CHUNK)
        big = bigs.at[b]
        cp0 = pltpu.async_copy(ptab0.at[iv0.at[sl]], big, gsem)
        cp0.wait()
        cp1 = pltpu.async_copy(ptab1.at[iv1.at[sl]], big, gsem, add=True)
        cp2 = pltpu.async_copy(ptab2.at[iv2.at[sl]], big, gsem, add=True)
        cp1.wait()
        cp2.wait()
        st = pltpu.async_copy(big, out.at[pl.ds(base + j * _CHUNK, _CHUNK)],
                              ssem)
        stores[b] = st
    for st in stores:
        if st is not None:
            st.wait()


def _sc_gather(ptab0, ptab1, ptab2, idx0, idx1, idx2, total):
    per_w = total // _NW
    mesh = plsc.VectorSubcoreMesh(
        core_axis_name="c", subcore_axis_name="s",
        num_cores=_NC, num_subcores=_NS,
    )
    run = pl.kernel(
        _gather_body,
        out_type=jax.ShapeDtypeStruct((total, _OUT_W), jnp.float32),
        mesh=mesh,
        scratch_types=[
            pltpu.VMEM((per_w,), jnp.int32),
            pltpu.VMEM((per_w,), jnp.int32),
            pltpu.VMEM((per_w,), jnp.int32),
            pltpu.VMEM((_NBUF, _CHUNK, _OUT_W), jnp.float32),
            pltpu.SemaphoreType.DMA,
            pltpu.SemaphoreType.DMA,
        ],
        compiler_params=pltpu.CompilerParams(use_tc_tiling_on_sc=False),
    )
    return run(ptab0, ptab1, ptab2, idx0, idx1, idx2)


def kernel(pos_ids, freqs0, freqs1, freqs2):
    B, N, _ = pos_ids.shape
    total = B * N

    ptab0, ptab1, ptab2 = _make_ptabs(freqs0, freqs1, freqs2)

    pos = pos_ids.astype(jnp.int32).reshape(total, 3)
    idxs = [
        jnp.clip(pos[:, a], 0, _AXES_LENS[a] - 1)
        for a in range(3)
    ]

    flat = _sc_gather(ptab0, ptab1, ptab2, *idxs, total)

    fc = lax.complex(flat[:, 0::2], flat[:, 1::2])
    return fc.reshape(B, N, _OUT_W // 2)


# speedup vs baseline: 2.9033x; 2.9033x over previous
import functools

import jax
import jax.numpy as jnp
from jax import lax
from jax.experimental import pallas as pl
from jax.experimental.pallas import tpu as pltpu
from jax.experimental.pallas import tpu_sc as plsc

_AXES_LENS = (1536, 512, 512)
_CW = (32, 48, 48)
_COFF = (0, 32, 80)
_OUT_W = 128

_NC = 2
_NS = 16
_NW = _NC * _NS
_CHUNK = 128
_NBUF = 2


def _tables_body(a0, a1, a2, o0, o1, o2):
    for a, o, off, w in zip((a0, a1, a2), (o0, o1, o2), _COFF, _CW):
        x = a[...]
        col = lax.broadcasted_iota(jnp.int32, x.shape, 1)
        val = jnp.where((col & 1) == 0, jnp.cos(x), jnp.sin(x))
        in_range = (col >= off) & (col < off + w)
        o[...] = jnp.where(in_range, val, 0.0)


def _make_ptabs(freqs0, freqs1, freqs2):
    reps = []
    for f, off, w in zip((freqs0, freqs1, freqs2), _COFF, _CW):
        r = jnp.repeat(f, 2, axis=1)
        reps.append(jnp.pad(r, ((0, 0), (off, _OUT_W - off - w))))
    out_shape = [
        jax.ShapeDtypeStruct((_AXES_LENS[i], _OUT_W), jnp.float32)
        for i in range(3)
    ]
    return pl.pallas_call(_tables_body, out_shape=out_shape)(*reps)


def _gather_body(ptab0, ptab1, ptab2, idx0, idx1, idx2, out,
                 iv0, iv1, iv2, bigs, gsem, ssem):
    wid = lax.axis_index("s") * _NC + lax.axis_index("c")
    per_w = iv0.shape[0]
    n_chunks = per_w // _CHUNK
    base = wid * per_w
    pltpu.sync_copy(idx0.at[pl.ds(base, per_w)], iv0)
    pltpu.sync_copy(idx1.at[pl.ds(base, per_w)], iv1)
    pltpu.sync_copy(idx2.at[pl.ds(base, per_w)], iv2)
    stores = [None] * _NBUF
    for j in range(n_chunks):
        b = j % _NBUF
        if stores[b] is not None:
            stores[b].wait()
        sl = pl.ds(j * _CHUNK, _CHUNK)
        big = bigs.at[b]
        cp0 = pltpu.async_copy(ptab0.at[iv0.at[sl]], big, gsem)
        cp0.wait()
        cp1 = pltpu.async_copy(ptab1.at[iv1.at[sl]], big, gsem, add=True)
        cp2 = pltpu.async_copy(ptab2.at[iv2.at[sl]], big, gsem, add=True)
        cp1.wait()
        cp2.wait()
        st = pltpu.async_copy(big, out.at[pl.ds(base + j * _CHUNK, _CHUNK)],
                              ssem)
        stores[b] = st
    for st in stores:
        if st is not None:
            st.wait()


def _sc_gather(ptab0, ptab1, ptab2, idx0, idx1, idx2, total):
    per_w = total // _NW
    mesh = plsc.VectorSubcoreMesh(
        core_axis_name="c", subcore_axis_name="s",
        num_cores=_NC, num_subcores=_NS,
    )
    run = pl.kernel(
        _gather_body,
        out_type=jax.ShapeDtypeStruct((total, _OUT_W), jnp.float32),
        mesh=mesh,
        scratch_types=[
            pltpu.VMEM((per_w,), jnp.int32),
            pltpu.VMEM((per_w,), jnp.int32),
            pltpu.VMEM((per_w,), jnp.int32),
            pltpu.VMEM((_NBUF, _CHUNK, _OUT_W), jnp.float32),
            pltpu.SemaphoreType.DMA,
            pltpu.SemaphoreType.DMA,
        ],
        compiler_params=pltpu.CompilerParams(use_tc_tiling_on_sc=False),
    )
    return run(ptab0, ptab1, ptab2, idx0, idx1, idx2)


def kernel(pos_ids, freqs0, freqs1, freqs2):
    B, N, _ = pos_ids.shape
    total = B * N

    ptab0, ptab1, ptab2 = _make_ptabs(freqs0, freqs1, freqs2)

    pos = pos_ids.astype(jnp.int32).reshape(total, 3)
    idxs = [
        jnp.clip(pos[:, a], 0, _AXES_LENS[a] - 1)
        for a in range(3)
    ]

    flat = _sc_gather(ptab0, ptab1, ptab2, *idxs, total)

    o = flat.reshape(B, N, _OUT_W // 2, 2)
    return lax.complex(o[..., 0], o[..., 1])


# speedup vs baseline: 3.1779x; 1.0946x over previous
import functools

import jax
import jax.numpy as jnp
from jax import lax
from jax.experimental import pallas as pl
from jax.experimental.pallas import tpu as pltpu
from jax.experimental.pallas import tpu_sc as plsc

_AXES_LENS = (1536, 512, 512)
_CW = (32, 48, 48)
_COFF = (0, 32, 80)
_OUT_W = 128

_NC = 2
_NS = 16
_NW = _NC * _NS
_CHUNK = 128
_NBUF = 2


def _tables_body(a0, a1, a2, o0, o1, o2):
    for a, o, off, w in zip((a0, a1, a2), (o0, o1, o2), _COFF, _CW):
        x = a[...]
        col = lax.broadcasted_iota(jnp.int32, x.shape, 1)
        val = jnp.where(col < _OUT_W // 2, jnp.cos(x), jnp.sin(x))
        c2 = col & (_OUT_W // 2 - 1)
        in_range = (c2 >= off // 2) & (c2 < (off + w) // 2)
        o[...] = jnp.where(in_range, val, 0.0)


def _make_ptabs(freqs0, freqs1, freqs2):
    reps = []
    for f, off, w in zip((freqs0, freqs1, freqs2), _COFF, _CW):
        half = jnp.pad(f, ((0, 0), (off // 2, (_OUT_W - off - w) // 2)))
        reps.append(jnp.concatenate([half, half], axis=1))
    out_shape = [
        jax.ShapeDtypeStruct((_AXES_LENS[i], _OUT_W), jnp.float32)
        for i in range(3)
    ]
    return pl.pallas_call(_tables_body, out_shape=out_shape)(*reps)


def _gather_body(ptab0, ptab1, ptab2, idx0, idx1, idx2, out,
                 iv0, iv1, iv2, bigs, gsem, ssem):
    wid = lax.axis_index("s") * _NC + lax.axis_index("c")
    per_w = iv0.shape[0]
    n_chunks = per_w // _CHUNK
    base = wid * per_w
    pltpu.sync_copy(idx0.at[pl.ds(base, per_w)], iv0)
    pltpu.sync_copy(idx1.at[pl.ds(base, per_w)], iv1)
    pltpu.sync_copy(idx2.at[pl.ds(base, per_w)], iv2)
    stores = [None] * _NBUF
    for j in range(n_chunks):
        b = j % _NBUF
        if stores[b] is not None:
            stores[b].wait()
        sl = pl.ds(j * _CHUNK, _CHUNK)
        big = bigs.at[b]
        cp0 = pltpu.async_copy(ptab0.at[iv0.at[sl]], big, gsem)
        cp0.wait()
        cp1 = pltpu.async_copy(ptab1.at[iv1.at[sl]], big, gsem, add=True)
        cp2 = pltpu.async_copy(ptab2.at[iv2.at[sl]], big, gsem, add=True)
        cp1.wait()
        cp2.wait()
        st = pltpu.async_copy(big, out.at[pl.ds(base + j * _CHUNK, _CHUNK)],
                              ssem)
        stores[b] = st
    for st in stores:
        if st is not None:
            st.wait()


def _sc_gather(ptab0, ptab1, ptab2, idx0, idx1, idx2, total):
    per_w = total // _NW
    mesh = plsc.VectorSubcoreMesh(
        core_axis_name="c", subcore_axis_name="s",
        num_cores=_NC, num_subcores=_NS,
    )
    run = pl.kernel(
        _gather_body,
        out_type=jax.ShapeDtypeStruct((total, _OUT_W), jnp.float32),
        mesh=mesh,
        scratch_types=[
            pltpu.VMEM((per_w,), jnp.int32),
            pltpu.VMEM((per_w,), jnp.int32),
            pltpu.VMEM((per_w,), jnp.int32),
            pltpu.VMEM((_NBUF, _CHUNK, _OUT_W), jnp.float32),
            pltpu.SemaphoreType.DMA,
            pltpu.SemaphoreType.DMA,
        ],
        compiler_params=pltpu.CompilerParams(use_tc_tiling_on_sc=True),
    )
    return run(ptab0, ptab1, ptab2, idx0, idx1, idx2)


def kernel(pos_ids, freqs0, freqs1, freqs2):
    B, N, _ = pos_ids.shape
    total = B * N

    ptab0, ptab1, ptab2 = _make_ptabs(freqs0, freqs1, freqs2)

    pos = pos_ids.astype(jnp.int32).reshape(total, 3)
    idxs = [
        jnp.clip(pos[:, a], 0, _AXES_LENS[a] - 1)
        for a in range(3)
    ]

    flat = _sc_gather(ptab0, ptab1, ptab2, *idxs, total)

    fc = lax.complex(flat[:, :_OUT_W // 2], flat[:, _OUT_W // 2:])
    return fc.reshape(B, N, _OUT_W // 2)


# speedup vs baseline: 3.2052x; 1.0086x over previous
import functools

import jax
import jax.numpy as jnp
from jax import lax
from jax.experimental import pallas as pl
from jax.experimental.pallas import tpu as pltpu
from jax.experimental.pallas import tpu_sc as plsc

_AXES_LENS = (1536, 512, 512)
_CW = (32, 48, 48)
_COFF = (0, 32, 80)
_OUT_W = 128

_NC = 2
_NS = 16
_NW = _NC * _NS
_CHUNK = 128
_NBUF = 2


def _tables_body(a0, a1, a2, o0, o1, o2):
    for a, o, off, w in zip((a0, a1, a2), (o0, o1, o2), _COFF, _CW):
        x = a[...]
        col = lax.broadcasted_iota(jnp.int32, x.shape, 1)
        val = jnp.where(col < _OUT_W // 2, jnp.cos(x), jnp.sin(x))
        c2 = col & (_OUT_W // 2 - 1)
        in_range = (c2 >= off // 2) & (c2 < (off + w) // 2)
        o[...] = jnp.where(in_range, val, 0.0)


def _make_ptabs(freqs0, freqs1, freqs2):
    reps = []
    for f, off, w in zip((freqs0, freqs1, freqs2), _COFF, _CW):
        half = jnp.pad(f, ((0, 0), (off // 2, (_OUT_W - off - w) // 2)))
        reps.append(jnp.concatenate([half, half], axis=1))
    out_shape = [
        jax.ShapeDtypeStruct((_AXES_LENS[i], _OUT_W), jnp.float32)
        for i in range(3)
    ]
    return pl.pallas_call(_tables_body, out_shape=out_shape)(*reps)


def _gather_body(ptab0, ptab1, ptab2, idx0, idx1, idx2, out,
                 iv0, iv1, iv2, bigs, gsem, ssem):
    wid = lax.axis_index("s") * _NC + lax.axis_index("c")
    per_w = iv0.shape[0]
    n_chunks = per_w // _CHUNK
    base = wid * per_w
    pltpu.sync_copy(idx0.at[pl.ds(base, per_w)], iv0)
    pltpu.sync_copy(idx1.at[pl.ds(base, per_w)], iv1)
    pltpu.sync_copy(idx2.at[pl.ds(base, per_w)], iv2)
    stores = [None] * _NBUF
    for j in range(n_chunks):
        b = j % _NBUF
        if stores[b] is not None:
            stores[b].wait()
        sl = pl.ds(j * _CHUNK, _CHUNK)
        big = bigs.at[b]
        cp0 = pltpu.async_copy(ptab0.at[iv0.at[sl]], big, gsem)
        cp0.wait()
        cp1 = pltpu.async_copy(ptab1.at[iv1.at[sl]], big, gsem, add=True)
        cp2 = pltpu.async_copy(ptab2.at[iv2.at[sl]], big, gsem, add=True)
        cp1.wait()
        cp2.wait()
        st = pltpu.async_copy(big, out.at[pl.ds(base + j * _CHUNK, _CHUNK)],
                              ssem)
        stores[b] = st
    for st in stores:
        if st is not None:
            st.wait()


def _sc_gather(ptab0, ptab1, ptab2, idx0, idx1, idx2, total):
    per_w = total // _NW
    mesh = plsc.VectorSubcoreMesh(
        core_axis_name="c", subcore_axis_name="s",
        num_cores=_NC, num_subcores=_NS,
    )
    run = pl.kernel(
        _gather_body,
        out_type=jax.ShapeDtypeStruct((total, _OUT_W), jnp.float32),
        mesh=mesh,
        scratch_types=[
            pltpu.VMEM((per_w,), jnp.int32),
            pltpu.VMEM((per_w,), jnp.int32),
            pltpu.VMEM((per_w,), jnp.int32),
            pltpu.VMEM((_NBUF, _CHUNK, _OUT_W), jnp.float32),
            pltpu.SemaphoreType.DMA,
            pltpu.SemaphoreType.DMA,
        ],
        compiler_params=pltpu.CompilerParams(use_tc_tiling_on_sc=True),
    )
    return run(ptab0, ptab1, ptab2, idx0, idx1, idx2)


_TBLK = 512


def _transpose_body(x_ref, re_ref, im_ref):
    x = x_ref[...]
    h = _OUT_W // 2
    re_ref[0] = jnp.transpose(x[:, :h], (1, 0))
    im_ref[0] = jnp.transpose(x[:, h:], (1, 0))


def _split_transpose(flat, B, N):
    h = _OUT_W // 2
    grid = (B * N // _TBLK,)
    nb = N // _TBLK
    out_shape = [jax.ShapeDtypeStruct((B, h, N), jnp.float32)] * 2
    return pl.pallas_call(
        _transpose_body,
        grid=grid,
        in_specs=[pl.BlockSpec((_TBLK, _OUT_W), lambda j: (j, 0))],
        out_specs=[
            pl.BlockSpec((1, h, _TBLK), lambda j: (j // nb, 0, j % nb)),
            pl.BlockSpec((1, h, _TBLK), lambda j: (j // nb, 0, j % nb)),
        ],
        out_shape=out_shape,
    )(flat)


def kernel(pos_ids, freqs0, freqs1, freqs2):
    B, N, _ = pos_ids.shape
    total = B * N

    ptab0, ptab1, ptab2 = _make_ptabs(freqs0, freqs1, freqs2)

    pos = pos_ids.astype(jnp.int32).reshape(total, 3)
    idxs = [
        jnp.clip(pos[:, a], 0, _AXES_LENS[a] - 1)
        for a in range(3)
    ]

    flat = _sc_gather(ptab0, ptab1, ptab2, *idxs, total)

    re_t, im_t = _split_transpose(flat, B, N)
    return lax.complex(
        jnp.transpose(re_t, (0, 2, 1)),
        jnp.transpose(im_t, (0, 2, 1)),
    )


# speedup vs baseline: 3.5294x; 1.1011x over previous
import functools

import jax
import jax.numpy as jnp
from jax import lax
from jax.experimental import pallas as pl
from jax.experimental.pallas import tpu as pltpu
from jax.experimental.pallas import tpu_sc as plsc

_AXES_LENS = (1536, 512, 512)
_CW = (32, 48, 48)
_COFF = (0, 32, 80)
_OUT_W = 128

_NC = 2
_NS = 16
_NW = _NC * _NS
_CHUNK = 128
_NBUF = 2


def _tables_body(a0, a1, a2, o0, o1, o2):
    for a, o, off, w in zip((a0, a1, a2), (o0, o1, o2), _COFF, _CW):
        x = a[...]
        col = lax.broadcasted_iota(jnp.int32, x.shape, 1)
        val = jnp.where(col < _OUT_W // 2, jnp.cos(x), jnp.sin(x))
        c2 = col & (_OUT_W // 2 - 1)
        in_range = (c2 >= off // 2) & (c2 < (off + w) // 2)
        o[...] = jnp.where(in_range, val, 0.0)


def _make_ptabs(freqs0, freqs1, freqs2):
    reps = []
    for f, off, w in zip((freqs0, freqs1, freqs2), _COFF, _CW):
        half = jnp.pad(f, ((0, 0), (off // 2, (_OUT_W - off - w) // 2)))
        reps.append(jnp.concatenate([half, half], axis=1))
    out_shape = [
        jax.ShapeDtypeStruct((_AXES_LENS[i], _OUT_W), jnp.float32)
        for i in range(3)
    ]
    return pl.pallas_call(_tables_body, out_shape=out_shape)(*reps)


def _gather_body(ptab0, ptab1, ptab2, idx0, idx1, idx2, out,
                 iv0, iv1, iv2, bigs, gsem, ssem):
    wid = lax.axis_index("s") * _NC + lax.axis_index("c")
    per_w = iv0.shape[0]
    n_chunks = per_w // _CHUNK
    base = wid * per_w
    pltpu.sync_copy(idx0.at[pl.ds(base, per_w)], iv0)
    pltpu.sync_copy(idx1.at[pl.ds(base, per_w)], iv1)
    pltpu.sync_copy(idx2.at[pl.ds(base, per_w)], iv2)
    stores = [None] * _NBUF
    for j in range(n_chunks):
        b = j % _NBUF
        if stores[b] is not None:
            stores[b].wait()
        sl = pl.ds(j * _CHUNK, _CHUNK)
        big = bigs.at[b]
        cp0 = pltpu.async_copy(ptab0.at[iv0.at[sl]], big, gsem)
        cp0.wait()
        cp1 = pltpu.async_copy(ptab1.at[iv1.at[sl]], big, gsem, add=True)
        cp2 = pltpu.async_copy(ptab2.at[iv2.at[sl]], big, gsem, add=True)
        cp1.wait()
        cp2.wait()
        st = pltpu.async_copy(big, out.at[pl.ds(base + j * _CHUNK, _CHUNK)],
                              ssem)
        stores[b] = st
    for st in stores:
        if st is not None:
            st.wait()


def _sc_gather(ptab0, ptab1, ptab2, idx0, idx1, idx2, total):
    per_w = total // _NW
    mesh = plsc.VectorSubcoreMesh(
        core_axis_name="c", subcore_axis_name="s",
        num_cores=_NC, num_subcores=_NS,
    )
    run = pl.kernel(
        _gather_body,
        out_type=jax.ShapeDtypeStruct((total, _OUT_W), jnp.float32),
        mesh=mesh,
        scratch_types=[
            pltpu.VMEM((per_w,), jnp.int32),
            pltpu.VMEM((per_w,), jnp.int32),
            pltpu.VMEM((per_w,), jnp.int32),
            pltpu.VMEM((_NBUF, _CHUNK, _OUT_W), jnp.float32),
            pltpu.SemaphoreType.DMA,
            pltpu.SemaphoreType.DMA,
        ],
        compiler_params=pltpu.CompilerParams(use_tc_tiling_on_sc=True),
    )
    return run(ptab0, ptab1, ptab2, idx0, idx1, idx2)


_TBLK = 2048


def _transpose_body(x_ref, re_ref, im_ref):
    x = x_ref[...]
    h = _OUT_W // 2
    re_ref[0] = jnp.transpose(x[:, :h], (1, 0))
    im_ref[0] = jnp.transpose(x[:, h:], (1, 0))


def _split_transpose(flat, B, N):
    h = _OUT_W // 2
    grid = (B * N // _TBLK,)
    nb = N // _TBLK
    out_shape = [jax.ShapeDtypeStruct((B, h, N), jnp.float32)] * 2
    return pl.pallas_call(
        _transpose_body,
        grid=grid,
        in_specs=[pl.BlockSpec((_TBLK, _OUT_W), lambda j: (j, 0))],
        out_specs=[
            pl.BlockSpec((1, h, _TBLK), lambda j: (j // nb, 0, j % nb)),
            pl.BlockSpec((1, h, _TBLK), lambda j: (j // nb, 0, j % nb)),
        ],
        out_shape=out_shape,
    )(flat)


def kernel(pos_ids, freqs0, freqs1, freqs2):
    B, N, _ = pos_ids.shape
    total = B * N

    ptab0, ptab1, ptab2 = _make_ptabs(freqs0, freqs1, freqs2)

    pos = pos_ids.astype(jnp.int32).reshape(total, 3)
    idxs = [
        jnp.clip(pos[:, a], 0, _AXES_LENS[a] - 1)
        for a in range(3)
    ]

    flat = _sc_gather(ptab0, ptab1, ptab2, *idxs, total)

    re_t, im_t = _split_transpose(flat, B, N)
    return lax.complex(
        jnp.transpose(re_t, (0, 2, 1)),
        jnp.transpose(im_t, (0, 2, 1)),
    )
